# baseline (speedup 1.0000x reference)
import functools

import jax
import jax.numpy as jnp
from jax import lax
from jax.experimental import pallas as pl
from jax.experimental.pallas import tpu as pltpu
from jax.experimental.pallas import tpu_sc as plsc

R = 8
K = 2 * R + 1
NCORES = 2
NSUB = 16
NW = NCORES * NSUB
GRP = 5


def _sc_body(qtab_hbm, out_hbm, qtab_v, buf, sems):
    H = out_hbm.shape[0]
    W = out_hbm.shape[1]
    rpr = W // 16
    n_img = H * W
    per_w = n_img // NW
    ngroups = per_w // GRP

    wid = lax.axis_index("s") * NCORES + lax.axis_index("c")
    img_base = wid * per_w

    pltpu.sync_copy(qtab_hbm, qtab_v)

    z16 = jnp.zeros((16,), jnp.float32)

    def zero_row(a, carry):
        for i in range(2 * GRP):
            for c in range(rpr):
                buf[i, a, pl.ds(16 * c, 16)] = z16
        return carry

    lax.fori_loop(0, H, zero_row, 0)

    def rows_img(s, slot_base, img0, write):
        img = img0 + s
        h1 = img // W
        w1 = img - h1 * W
        a0 = h1 - R
        b0 = w1 - R
        d0 = pl.multiple_of(16 * jnp.clip(b0 // 16, 0, rpr - 1), 16)
        d1 = pl.multiple_of(16 * jnp.clip(b0 // 16 + 1, 0, rpr - 1), 16)
        kh_lo = jnp.maximum(0, -a0)
        kh_hi = jnp.minimum(K, H - a0)
        slot_img = slot_base + s
        qbase = w1 * (2 * K)

        def row(kh, carry):
            a = a0 + kh
            if write:
                buf[slot_img, a, pl.ds(d0, 16)] = qtab_v[qbase + 2 * kh]
                buf[slot_img, a, pl.ds(d1, 16)] = qtab_v[qbase + 2 * kh + 1]
            else:
                buf[slot_img, a, pl.ds(d0, 16)] = z16
                buf[slot_img, a, pl.ds(d1, 16)] = z16
            return carry

        lax.fori_loop(kh_lo, kh_hi, row, 0)

    def group(g, carry):
        slot = lax.rem(g, 2)
        slot_base = slot * GRP
        img0 = img_base + g * GRP
        h1g = img0 // W
        w1g = img0 - h1g * W

        @pl.when(g >= 2)
        def _():
            pltpu.make_async_copy(
                buf.at[pl.ds(0, GRP)],
                out_hbm.at[0, pl.ds(0, GRP)],
                sems.at[slot],
            ).wait()

            def clear_one(s, c):
                rows_img(s, slot_base, img0 - 2 * GRP, False)
                return c

            lax.fori_loop(0, GRP, clear_one, 0)

        def write_one(s, c):
            rows_img(s, slot_base, img0, True)
            return c

        lax.fori_loop(0, GRP, write_one, 0)

        pltpu.make_async_copy(
            buf.at[pl.ds(slot_base, GRP)],
            out_hbm.at[h1g, pl.ds(w1g, GRP)],
            sems.at[slot],
        ).start()
        return carry

    lax.fori_loop(0, ngroups, group, 0)

    for slot in range(2):
        pltpu.make_async_copy(
            buf.at[pl.ds(0, GRP)],
            out_hbm.at[0, pl.ds(0, GRP)],
            sems.at[slot],
        ).wait()


def kernel(feat, biases, all_h1s, all_w1s, all_h2s, all_w2s):
    H, W = feat.shape[-2], feat.shape[-1]
    OFFP = 71
    ptab = jnp.zeros((K, 160), jnp.float32)
    ptab = lax.dynamic_update_slice(ptab, biases.astype(jnp.float32), (0, OFFP))
    w1 = jnp.arange(W, dtype=jnp.int32)
    b0 = w1 - R
    d0 = jnp.clip(b0 // 16, 0, W // 16 - 1)
    d1 = jnp.clip(b0 // 16 + 1, 0, W // 16 - 1)
    s01 = jnp.stack([16 * d0 - b0 + OFFP, 16 * d1 - b0 + OFFP], axis=1)
    cols = s01[:, :, None] + jnp.arange(16, dtype=jnp.int32)
    qtab = ptab[:, cols]
    qtab = jnp.transpose(qtab, (1, 0, 2, 3)).reshape(W * K * 2, 16)

    sc_kernel = functools.partial(
        pl.kernel,
        out_type=jax.ShapeDtypeStruct((H, W, H, W), jnp.float32),
        mesh=plsc.VectorSubcoreMesh(core_axis_name="c", subcore_axis_name="s"),
        scratch_types=[
            pltpu.VMEM((W * K * 2, 16), jnp.float32),
            pltpu.VMEM((2 * GRP, H, W), jnp.float32),
            pltpu.SemaphoreType.DMA((2,)),
        ],
        compiler_params=pltpu.CompilerParams(use_tc_tiling_on_sc=False),
    )(_sc_body)

    out = sc_kernel(qtab)
    return out[None, None]


# speedup vs baseline: 1.0001x; 1.0001x over previous
import functools

import jax
import jax.numpy as jnp
from jax import lax
from jax.experimental import pallas as pl
from jax.experimental.pallas import tpu as pltpu
from jax.experimental.pallas import tpu_sc as plsc

R = 8
K = 2 * R + 1
NCORES = 2
NSUB = 16
NW = NCORES * NSUB
GRP = 5


def _sc_body(qtab_hbm, out_hbm, qtab_v, buf, sems):
    H = out_hbm.shape[2]
    W = out_hbm.shape[3]
    rpr = W // 16
    n_img = H * W
    per_w = n_img // NW
    ngroups = per_w // GRP

    wid = lax.axis_index("s") * NCORES + lax.axis_index("c")
    img_base = wid * per_w

    pltpu.sync_copy(qtab_hbm, qtab_v)

    z16 = jnp.zeros((16,), jnp.float32)

    def zero_row(a, carry):
        for i in range(2 * GRP):
            for c in range(rpr):
                buf[i, a, pl.ds(16 * c, 16)] = z16
        return carry

    lax.fori_loop(0, H, zero_row, 0)

    def rows_img(s, slot_base, img0, write):
        img = img0 + s
        h1 = img // W
        w1 = img - h1 * W
        a0 = h1 - R
        b0 = w1 - R
        d0 = pl.multiple_of(16 * jnp.clip(b0 // 16, 0, rpr - 1), 16)
        d1 = pl.multiple_of(16 * jnp.clip(b0 // 16 + 1, 0, rpr - 1), 16)
        kh_lo = jnp.maximum(0, -a0)
        kh_hi = jnp.minimum(K, H - a0)
        slot_img = slot_base + s
        qbase = w1 * (2 * K)

        def row(kh, carry):
            a = a0 + kh
            if write:
                buf[slot_img, a, pl.ds(d0, 16)] = qtab_v[qbase + 2 * kh]
                buf[slot_img, a, pl.ds(d1, 16)] = qtab_v[qbase + 2 * kh + 1]
            else:
                buf[slot_img, a, pl.ds(d0, 16)] = z16
                buf[slot_img, a, pl.ds(d1, 16)] = z16
            return carry

        lax.fori_loop(kh_lo, kh_hi, row, 0)

    def group(g, carry):
        slot = lax.rem(g, 2)
        slot_base = slot * GRP
        img0 = img_base + g * GRP
        h1g = img0 // W
        w1g = img0 - h1g * W

        @pl.when(g >= 2)
        def _():
            pltpu.make_async_copy(
                buf.at[pl.ds(0, GRP)],
                out_hbm.at[0, 0, 0, pl.ds(0, GRP)],
                sems.at[slot],
            ).wait()

            def clear_one(s, c):
                rows_img(s, slot_base, img0 - 2 * GRP, False)
                return c

            lax.fori_loop(0, GRP, clear_one, 0)

        def write_one(s, c):
            rows_img(s, slot_base, img0, True)
            return c

        lax.fori_loop(0, GRP, write_one, 0)

        pltpu.make_async_copy(
            buf.at[pl.ds(slot_base, GRP)],
            out_hbm.at[0, 0, h1g, pl.ds(w1g, GRP)],
            sems.at[slot],
        ).start()
        return carry

    lax.fori_loop(0, ngroups, group, 0)

    for slot in range(2):
        pltpu.make_async_copy(
            buf.at[pl.ds(0, GRP)],
            out_hbm.at[0, 0, 0, pl.ds(0, GRP)],
            sems.at[slot],
        ).wait()


def kernel(feat, biases, all_h1s, all_w1s, all_h2s, all_w2s):
    H, W = feat.shape[-2], feat.shape[-1]
    OFFP = 71
    ptab = jnp.zeros((K, 160), jnp.float32)
    ptab = lax.dynamic_update_slice(ptab, biases.astype(jnp.float32), (0, OFFP))
    w1 = jnp.arange(W, dtype=jnp.int32)
    b0 = w1 - R
    d0 = jnp.clip(b0 // 16, 0, W // 16 - 1)
    d1 = jnp.clip(b0 // 16 + 1, 0, W // 16 - 1)
    s01 = jnp.stack([16 * d0 - b0 + OFFP, 16 * d1 - b0 + OFFP], axis=1)
    cols = s01[:, :, None] + jnp.arange(16, dtype=jnp.int32)
    qtab = ptab[:, cols]
    qtab = jnp.transpose(qtab, (1, 0, 2, 3)).reshape(W * K * 2, 16)

    sc_kernel = functools.partial(
        pl.kernel,
        out_type=jax.ShapeDtypeStruct((1, 1, H, W, H, W), jnp.float32),
        mesh=plsc.VectorSubcoreMesh(core_axis_name="c", subcore_axis_name="s"),
        scratch_types=[
            pltpu.VMEM((W * K * 2, 16), jnp.float32),
            pltpu.VMEM((2 * GRP, H, W), jnp.float32),
            pltpu.SemaphoreType.DMA((2,)),
        ],
        compiler_params=pltpu.CompilerParams(use_tc_tiling_on_sc=False),
    )(_sc_body)

    return sc_kernel(qtab)


# speedup vs baseline: 2.7923x; 2.7921x over previous
import functools

import jax
import jax.numpy as jnp
from jax import lax
from jax.experimental import pallas as pl
from jax.experimental.pallas import tpu as pltpu
from jax.experimental.pallas import tpu_sc as plsc

R = 8
K = 2 * R + 1
NCORES = 2
NSUB = 16
NW = NCORES * NSUB
GRP = 4
NQROW = 2 * K * 80 // 8


def _sc_body(qtab_hbm, out_hbm, qtab_v, buf, sems):
    H = out_hbm.shape[2]
    W = out_hbm.shape[3]
    rpr = W // 16
    n_img = H * W
    per_w = n_img // NW
    ngroups = per_w // GRP

    wid = lax.axis_index("s") * NCORES + lax.axis_index("c")
    img_base = wid * per_w

    pltpu.sync_copy(qtab_hbm, qtab_v)

    z16 = jnp.zeros((16,), jnp.float32)

    def zero_row(a, carry):
        for i in range(2 * GRP):
            for c in range(rpr):
                buf[i, a, pl.ds(16 * c, 16)] = z16
        return carry

    lax.fori_loop(0, H, zero_row, 0)

    def rows_img(s, slot_base, img0, write):
        img = img0 + s
        h1 = img // W
        w1 = img - h1 * W
        a0 = h1 - R
        b0 = w1 - R
        d0 = pl.multiple_of(16 * jnp.clip(b0 // 16, 0, rpr - 1), 16)
        d1 = pl.multiple_of(16 * jnp.clip(b0 // 16 + 1, 0, rpr - 1), 16)
        kh_lo = jnp.maximum(0, -a0)
        kh_hi = jnp.minimum(K, H - a0)
        slot_img = slot_base + s
        qbase = w1 * (2 * K)

        def row(kh, carry):
            a = a0 + kh
            q0 = qbase + 2 * kh
            q1 = q0 + 1
            if write:
                buf[slot_img, a, pl.ds(d0, 16)] = qtab_v[
                    q0 // 8, pl.ds(pl.multiple_of(16 * lax.rem(q0, 8), 16), 16)
                ]
                buf[slot_img, a, pl.ds(d1, 16)] = qtab_v[
                    q1 // 8, pl.ds(pl.multiple_of(16 * lax.rem(q1, 8), 16), 16)
                ]
            else:
                buf[slot_img, a, pl.ds(d0, 16)] = z16
                buf[slot_img, a, pl.ds(d1, 16)] = z16
            return carry

        lax.fori_loop(kh_lo, kh_hi, row, 0)

    def group(g, carry):
        slot = lax.rem(g, 2)
        slot_base = slot * GRP
        img0 = img_base + g * GRP
        h1g = img0 // W
        w1g = img0 - h1g * W

        @pl.when(g >= 2)
        def _():
            pltpu.make_async_copy(
                buf.at[pl.ds(0, GRP)],
                out_hbm.at[0, 0, 0, pl.ds(0, GRP)],
                sems.at[slot],
            ).wait()

            def clear_one(s, c):
                rows_img(s, slot_base, img0 - 2 * GRP, False)
                return c

            lax.fori_loop(0, GRP, clear_one, 0)

        def write_one(s, c):
            rows_img(s, slot_base, img0, True)
            return c

        lax.fori_loop(0, GRP, write_one, 0)

        pltpu.make_async_copy(
            buf.at[pl.ds(slot_base, GRP)],
            out_hbm.at[0, 0, h1g, pl.ds(w1g, GRP)],
            sems.at[slot],
        ).start()
        return carry

    lax.fori_loop(0, ngroups, group, 0)

    for slot in range(2):
        pltpu.make_async_copy(
            buf.at[pl.ds(0, GRP)],
            out_hbm.at[0, 0, 0, pl.ds(0, GRP)],
            sems.at[slot],
        ).wait()


def kernel(feat, biases, all_h1s, all_w1s, all_h2s, all_w2s):
    H, W = feat.shape[-2], feat.shape[-1]
    OFFP = 71
    ptab = jnp.zeros((K, 160), jnp.float32)
    ptab = lax.dynamic_update_slice(ptab, biases.astype(jnp.float32), (0, OFFP))
    w1 = jnp.arange(W, dtype=jnp.int32)
    b0 = w1 - R
    d0 = jnp.clip(b0 // 16, 0, W // 16 - 1)
    d1 = jnp.clip(b0 // 16 + 1, 0, W // 16 - 1)
    s01 = jnp.stack([16 * d0 - b0 + OFFP, 16 * d1 - b0 + OFFP], axis=1)
    cols = s01[:, :, None] + jnp.arange(16, dtype=jnp.int32)
    qtab = ptab[:, cols]
    qtab = jnp.transpose(qtab, (1, 0, 2, 3)).reshape(NQROW, 128)

    sc_kernel = functools.partial(
        pl.kernel,
        out_type=jax.ShapeDtypeStruct((1, 1, H, W, H, W), jnp.float32),
        mesh=plsc.VectorSubcoreMesh(core_axis_name="c", subcore_axis_name="s"),
        scratch_types=[
            pltpu.VMEM((NQROW, 128), jnp.float32),
            pltpu.VMEM((2 * GRP, H, W), jnp.float32),
            pltpu.SemaphoreType.DMA((2,)),
        ],
        compiler_params=pltpu.CompilerParams(use_tc_tiling_on_sc=True),
    )(_sc_body)

    return sc_kernel(qtab)


# speedup vs baseline: 2.9554x; 1.0584x over previous
import functools

import jax
import jax.numpy as jnp
from jax import lax
from jax.experimental import pallas as pl
from jax.experimental.pallas import tpu as pltpu
from jax.experimental.pallas import tpu_sc as plsc

R = 8
K = 2 * R + 1
NCORES = 2
NSUB = 16
NW = NCORES * NSUB
GRP = 4
NQROW = 2 * K * 80 // 8


def _sc_body(qtab_hbm, out_hbm, qtab_v, buf, sems):
    H = out_hbm.shape[2]
    W = out_hbm.shape[3]
    rpr = W // 16
    n_img = H * W
    per_w = n_img // NW
    ngroups = per_w // GRP

    wid = lax.axis_index("s") * NCORES + lax.axis_index("c")
    img_base = wid * per_w

    pltpu.sync_copy(qtab_hbm, qtab_v)

    z16 = jnp.zeros((16,), jnp.float32)

    def zero_row(a, carry):
        for i in range(2 * GRP):
            for c in range(rpr):
                buf[i, a, pl.ds(16 * c, 16)] = z16
        return carry

    lax.fori_loop(0, H, zero_row, 0)

    def rows_img(s, slot_base, img0, write):
        img = img0 + s
        h1 = img // W
        w1 = img - h1 * W
        a0 = h1 - R
        b0 = w1 - R
        d0 = pl.multiple_of(16 * jnp.clip(b0 // 16, 0, rpr - 1), 16)
        d1 = pl.multiple_of(16 * jnp.clip(b0 // 16 + 1, 0, rpr - 1), 16)
        kh_lo = jnp.maximum(0, -a0)
        kh_hi = jnp.minimum(K, H - a0)
        slot_img = slot_base + s
        qbase = w1 * (2 * K)

        def row(kh, carry):
            a = a0 + kh
            q0 = qbase + 2 * kh
            q1 = q0 + 1
            if write:
                buf[slot_img, a, pl.ds(d0, 16)] = qtab_v[
                    q0 // 8, pl.ds(pl.multiple_of(16 * lax.rem(q0, 8), 16), 16)
                ]
                buf[slot_img, a, pl.ds(d1, 16)] = qtab_v[
                    q1 // 8, pl.ds(pl.multiple_of(16 * lax.rem(q1, 8), 16), 16)
                ]
            else:
                buf[slot_img, a, pl.ds(d0, 16)] = z16
                buf[slot_img, a, pl.ds(d1, 16)] = z16
            return carry

        lax.fori_loop(kh_lo, kh_hi, row, 0)

    def group(g, carry):
        slot = lax.rem(g, 2)
        slot_base = slot * GRP
        img0 = img_base + g * GRP
        h1g = img0 // W
        w1g = img0 - h1g * W

        @pl.when(g >= 2)
        def _():
            pltpu.make_async_copy(
                buf.at[pl.ds(0, GRP)],
                out_hbm.at[0, 0, 0, pl.ds(0, GRP)],
                sems.at[slot],
            ).wait()

            def clear_one(s, c):
                rows_img(s, slot_base, img0 - 2 * GRP, False)
                return c

            lax.fori_loop(0, GRP, clear_one, 0)

        def write_one(s, c):
            rows_img(s, slot_base, img0, True)
            return c

        lax.fori_loop(0, GRP, write_one, 0)

        pltpu.make_async_copy(
            buf.at[pl.ds(slot_base, GRP)],
            out_hbm.at[0, 0, h1g, pl.ds(w1g, GRP)],
            sems.at[slot],
        ).start()
        return carry

    lax.fori_loop(0, ngroups, group, 0)

    for slot in range(2):
        pltpu.make_async_copy(
            buf.at[pl.ds(0, GRP)],
            out_hbm.at[0, 0, 0, pl.ds(0, GRP)],
            sems.at[slot],
        ).wait()


def kernel(feat, biases, all_h1s, all_w1s, all_h2s, all_w2s):
    H, W = feat.shape[-2], feat.shape[-1]
    w1 = jnp.arange(W, dtype=jnp.int32)
    b0 = w1 - R
    d0 = jnp.clip(b0 // 16, 0, W // 16 - 1)
    d1 = jnp.clip(b0 // 16 + 1, 0, W // 16 - 1)
    s01 = jnp.stack([16 * d0, 16 * d1], axis=1)
    kw = s01[:, :, None] + jnp.arange(16, dtype=jnp.int32) - b0[:, None, None]
    sel = (kw[..., None] == jnp.arange(K, dtype=jnp.int32)).astype(jnp.float32)
    qtab = jnp.einsum(
        "wclk,hk->whcl", sel, biases.astype(jnp.float32), preferred_element_type=jnp.float32
    ).reshape(NQROW, 128)

    sc_kernel = functools.partial(
        pl.kernel,
        out_type=jax.ShapeDtypeStruct((1, 1, H, W, H, W), jnp.float32),
        mesh=plsc.VectorSubcoreMesh(core_axis_name="c", subcore_axis_name="s"),
        scratch_types=[
            pltpu.VMEM((NQROW, 128), jnp.float32),
            pltpu.VMEM((2 * GRP, H, W), jnp.float32),
            pltpu.SemaphoreType.DMA((2,)),
        ],
        compiler_params=pltpu.CompilerParams(use_tc_tiling_on_sc=True),
    )(_sc_body)

    return sc_kernel(qtab)


# speedup vs baseline: 2.9582x; 1.0010x over previous
import functools

import jax
import jax.numpy as jnp
from jax import lax
from jax.experimental import pallas as pl
from jax.experimental.pallas import tpu as pltpu
from jax.experimental.pallas import tpu_sc as plsc

R = 8
K = 2 * R + 1
NCORES = 2
NSUB = 16
NW = NCORES * NSUB
GRP = 4
NQROW = 2 * K * 80 // 8


def _sc_body(qtab_hbm, out_hbm, qtab_v, buf, sems):
    H = out_hbm.shape[2]
    W = out_hbm.shape[3]
    rpr = W // 16
    n_img = H * W
    per_w = n_img // NW
    ngroups = per_w // GRP

    wid = lax.axis_index("s") * NCORES + lax.axis_index("c")
    img_base = wid * per_w

    pltpu.sync_copy(qtab_hbm, qtab_v)

    z16 = jnp.zeros((16,), jnp.float32)

    def zero_row(a, carry):
        for i in range(2 * GRP):
            for c in range(rpr):
                buf[i, a, pl.ds(16 * c, 16)] = z16
        return carry

    lax.fori_loop(0, H, zero_row, 0)

    def rows_img(s, slot_base, img0, write):
        img = img0 + s
        h1 = img // W
        w1 = img - h1 * W
        a0 = h1 - R
        b0 = w1 - R
        d0 = pl.multiple_of(16 * jnp.clip(b0 // 16, 0, rpr - 1), 16)
        d1 = pl.multiple_of(16 * jnp.clip(b0 // 16 + 1, 0, rpr - 1), 16)
        kh_lo = jnp.maximum(0, -a0)
        kh_hi = jnp.minimum(K, H - a0)
        slot_img = slot_base + s
        qbase = w1 * (2 * K)

        def row(kh, carry):
            a = a0 + kh
            q0 = qbase + 2 * kh
            q1 = q0 + 1
            if write:
                buf[slot_img, a, pl.ds(d0, 16)] = qtab_v[
                    q0 // 8, pl.ds(pl.multiple_of(16 * lax.rem(q0, 8), 16), 16)
                ]
                buf[slot_img, a, pl.ds(d1, 16)] = qtab_v[
                    q1 // 8, pl.ds(pl.multiple_of(16 * lax.rem(q1, 8), 16), 16)
                ]
            else:
                buf[slot_img, a, pl.ds(d0, 16)] = z16
                buf[slot_img, a, pl.ds(d1, 16)] = z16
            return carry

        lax.fori_loop(kh_lo, kh_hi, row, 0)

    def group(g, carry):
        slot = lax.rem(g, 2)
        slot_base = slot * GRP
        img0 = img_base + g * GRP
        h1g = img0 // W
        w1g = img0 - h1g * W

        @pl.when(g >= 2)
        def _():
            pltpu.make_async_copy(
                buf.at[pl.ds(0, GRP)],
                out_hbm.at[0, 0, 0, pl.ds(0, GRP)],
                sems.at[slot],
            ).wait()

            def clear_one(s, c):
                rows_img(s, slot_base, img0 - 2 * GRP, False)
                return c

            lax.fori_loop(0, GRP, clear_one, 0)

        def write_one(s, c):
            rows_img(s, slot_base, img0, True)
            return c

        lax.fori_loop(0, GRP, write_one, 0)

        pltpu.make_async_copy(
            buf.at[pl.ds(slot_base, GRP)],
            out_hbm.at[0, 0, h1g, pl.ds(w1g, GRP)],
            sems.at[slot],
        ).start()
        return carry

    lax.fori_loop(0, ngroups, group, 0)

    for slot in range(2):
        pltpu.make_async_copy(
            buf.at[pl.ds(0, GRP)],
            out_hbm.at[0, 0, 0, pl.ds(0, GRP)],
            sems.at[slot],
        ).wait()


def kernel(feat, biases, all_h1s, all_w1s, all_h2s, all_w2s):
    H, W = feat.shape[-2], feat.shape[-1]
    w1 = jnp.arange(W, dtype=jnp.int32)
    b0 = w1 - R
    d0 = jnp.clip(b0 // 16, 0, W // 16 - 1)
    d1 = jnp.clip(b0 // 16 + 1, 0, W // 16 - 1)
    s01 = jnp.stack([16 * d0, 16 * d1], axis=1)
    kw = s01[:, :, None] + jnp.arange(16, dtype=jnp.int32) - b0[:, None, None]
    sel = (kw[..., None] == jnp.arange(K, dtype=jnp.int32)).astype(jnp.float32)
    qtab = jnp.einsum(
        "wclk,hk->whcl",
        sel,
        biases.astype(jnp.float32),
        preferred_element_type=jnp.float32,
        precision=lax.Precision.HIGHEST,
    ).reshape(NQROW, 128)

    sc_kernel = functools.partial(
        pl.kernel,
        out_type=jax.ShapeDtypeStruct((1, 1, H, W, H, W), jnp.float32),
        mesh=plsc.VectorSubcoreMesh(core_axis_name="c", subcore_axis_name="s"),
        scratch_types=[
            pltpu.VMEM((NQROW, 128), jnp.float32),
            pltpu.VMEM((2 * GRP, H, W), jnp.float32),
            pltpu.SemaphoreType.DMA((2,)),
        ],
        compiler_params=pltpu.CompilerParams(use_tc_tiling_on_sc=True),
    )(_sc_body)

    return sc_kernel(qtab)
